# baseline (speedup 1.0000x reference)
import functools

import jax
import jax.numpy as jnp
from jax import lax
from jax.experimental import pallas as pl
from jax.experimental.pallas import tpu as pltpu
from jax.experimental.pallas import tpu_sc as plsc

B, F, D = 16384, 26, 32
CARD = 100000
ROWS_W = (CARD * F * D) // 128
NW = 32
BW = B // NW
SB = 128
NSB = BW // SB
KV = SB // 16


@functools.lru_cache(maxsize=1)
def _build():
    mesh = plsc.VectorSubcoreMesh(core_axis_name="c", subcore_axis_name="s")
    return functools.partial(
        pl.kernel,
        out_type=jax.ShapeDtypeStruct((F, D, B), jnp.float32),
        mesh=mesh,
        scratch_types=[
            pltpu.VMEM((BW,), jnp.int32),
            pltpu.VMEM((SB,), jnp.int32),
            pltpu.VMEM((SB,), jnp.int32),
            pltpu.VMEM((SB, 128), jnp.float32),
            pltpu.VMEM((D, SB), jnp.float32),
            pltpu.VMEM((F * D,), jnp.float32),
            pltpu.VMEM((F * D * 16,), jnp.float32),
            pltpu.SemaphoreType.DMA,
        ],
        compiler_params=pltpu.CompilerParams(needs_layout_passes=False),
    )(_embed_body)


def _embed_body(xtf_hbm, tablew_hbm, shared_hbm, out_hbm,
                xf_v, widx_v, sub_v, wide_v, outb_v, sh_v, shb_v, sem):
    wid = lax.axis_index("s") * 2 + lax.axis_index("c")
    b0 = wid * BW

    pltpu.sync_copy(shared_hbm, sh_v)

    def splat_body(j, carry):
        e = plsc.load_gather(sh_v, [jnp.full((16,), j, jnp.int32)])
        shb_v[pl.ds(j * 16, 16)] = e
        return carry

    lax.fori_loop(0, F * D, splat_body, 0)

    def field_body(f, carry):
        off = f * CARD
        pltpu.sync_copy(xtf_hbm.at[pl.ds(f * B + b0, BW)], xf_v)

        for sb in range(NSB):
            for k in range(KV):
                v = xf_v[pl.ds(sb * SB + k * 16, 16)] + off
                widx_v[pl.ds(k * 16, 16)] = v >> 2
                sub_v[pl.ds(k * 16, 16)] = (v & 3) * D

            pltpu.async_copy(tablew_hbm.at[widx_v], wide_v, sem).wait()

            def extract_body(k, carry2):
                rows = lax.iota(jnp.int32, 16) + k * 16
                cols = sub_v[pl.ds(k * 16, 16)]
                for d in range(D):
                    e = plsc.load_gather(wide_v, [rows, cols + d])
                    o = e + shb_v[pl.ds((f * D + d) * 16, 16)]
                    outb_v[d, pl.ds(k * 16, 16)] = o
                return carry2

            lax.fori_loop(0, KV, extract_body, 0)

            pltpu.sync_copy(outb_v, out_hbm.at[f, :, pl.ds(b0 + sb * SB, SB)])
        return carry

    lax.fori_loop(0, F, field_body, 0)


def kernel(x, indiv_embed, shared_embed):
    xtf = x.T.reshape(F * B)
    tw = indiv_embed.reshape(ROWS_W, 128)
    outp = _build()(xtf, tw, shared_embed.reshape(F * D))
    return jnp.transpose(outp, (2, 0, 1))
